# baseline (speedup 1.0000x reference)
import functools

import jax
import jax.numpy as jnp
from jax import lax
from jax.experimental import pallas as pl
from jax.experimental.pallas import tpu as pltpu
from jax.experimental.pallas import tpu_sc as plsc

_info = plsc.get_sparse_core_info()
_NC, _NS = _info.num_cores, _info.num_subcores
_NW = _NC * _NS

_V = 1_000_000
_D = 64
_G_FULL = _V // 128
_WR_ROWS = _V // 2


def _iota16():
    return lax.iota(jnp.int32, 16)


def _transpose_pairs(src, dst, nk):
    rows = [_iota16() + 16 * jj for jj in range(4)]

    def krow(k, carry):
        c0 = 2 * k
        ca = jnp.full((16,), c0, jnp.int32)
        cb = jnp.full((16,), c0 + 1, jnp.int32)
        for j in range(8):
            v = plsc.load_gather(src, [rows[j % 4], ca if j < 4 else cb])
            dst[k, pl.ds(16 * j, 16)] = v
        return carry

    lax.fori_loop(0, nk, krow, 0)


def _make_table_transpose():
    mesh = plsc.VectorSubcoreMesh(core_axis_name="c", subcore_axis_name="s")

    @functools.partial(
        pl.kernel,
        mesh=mesh,
        out_type=jax.ShapeDtypeStruct((_WR_ROWS, 128), jnp.float32),
        scratch_types=[
            pltpu.VMEM((2, _D, 128), jnp.float32),
            pltpu.VMEM((2, _D, 128), jnp.float32),
            pltpu.VMEM((_D, _D), jnp.float32),
            pltpu.VMEM((32, 128), jnp.float32),
            pltpu.SemaphoreType.DMA,
            pltpu.SemaphoreType.DMA,
            pltpu.SemaphoreType.DMA,
            pltpu.SemaphoreType.DMA,
        ],
        compiler_params=pltpu.CompilerParams(needs_layout_passes=False),
    )
    def k1(wt_hbm, wr_hbm, a_in, a_out, t_in, t_out, si0, si1, so0, so1):
        wid = lax.axis_index("s") * _NC + lax.axis_index("c")
        n_g = jnp.where(wid < 2, 246, 244)
        g0 = 244 * wid + 2 * jnp.minimum(wid, 2)
        si = (si0, si1)
        so = (so0, so1)

        def fire_in(t, b):
            pltpu.async_copy(
                wt_hbm.at[:, pl.ds(128 * (g0 + t), 128)], a_in.at[b], si[b]
            )

        def wait_in(t, b):
            pltpu.make_async_copy(
                wt_hbm.at[:, pl.ds(128 * (g0 + t), 128)], a_in.at[b], si[b]
            ).wait()

        def fire_out(t, b):
            pltpu.async_copy(
                a_out.at[b], wr_hbm.at[pl.ds(64 * (g0 + t), 64)], so[b]
            )

        def wait_out(t, b):
            pltpu.make_async_copy(
                a_out.at[b], wr_hbm.at[pl.ds(64 * (g0 + t), 64)], so[b]
            ).wait()

        fire_in(0, 0)
        fire_in(1, 1)
        for b in (0, 1):
            wait_in(b, b)
            _transpose_pairs(a_in.at[b], a_out.at[b], 64)
            fire_out(b, b)
            fire_in(b + 2, b)

        def body(p, carry):
            for b in (0, 1):
                t = 2 * p + b
                wait_in(t, b)
                wait_out(t - 2, b)
                _transpose_pairs(a_in.at[b], a_out.at[b], 64)
                fire_out(t, b)
                fire_in(t + 2, b)
            return carry

        lax.fori_loop(1, n_g // 2 - 1, body, 0)

        for b in (0, 1):
            t = n_g - 2 + b
            wait_in(t, b)
            wait_out(t - 2, b)
            _transpose_pairs(a_in.at[b], a_out.at[b], 64)
            fire_out(t, b)
        wait_out(n_g - 2, 0)
        wait_out(n_g - 1, 1)

        @pl.when(wid == _NW - 1)
        def _tail():
            pltpu.sync_copy(wt_hbm.at[:, pl.ds(128 * _G_FULL, 64)], t_in)
            _transpose_pairs(t_in, t_out, 32)
            pltpu.sync_copy(t_out, wr_hbm.at[pl.ds(64 * _G_FULL, 32)])

    return k1


def _make_lookup(n_s: int):
    mesh = plsc.VectorSubcoreMesh(core_axis_name="c", subcore_axis_name="s")
    b_cols = 4096 // 128

    @functools.partial(
        pl.kernel,
        mesh=mesh,
        out_type=jax.ShapeDtypeStruct((n_s, _D, 4096), jnp.float32),
        scratch_types=[
            pltpu.VMEM((n_s, 128), jnp.int32),
            pltpu.VMEM((2, 128, 128), jnp.float32),
            pltpu.VMEM((2, _D, 129), jnp.float32),
            pltpu.SemaphoreType.DMA,
            pltpu.SemaphoreType.DMA,
            pltpu.SemaphoreType.DMA,
            pltpu.SemaphoreType.DMA,
        ],
        compiler_params=pltpu.CompilerParams(needs_layout_passes=False),
    )
    def k2(idx_hbm, wr_hbm, out_hbm, slab, tg, ob, sg0, sg1, so0, so1):
        wid = lax.axis_index("s") * _NC + lax.axis_index("c")
        n_blk = n_s
        sg = (sg0, sg1)
        so = (so0, so1)

        pltpu.sync_copy(idx_hbm.at[:, pl.ds(128 * wid, 128)], slab)

        def fire_g(t, b):
            pltpu.async_copy(wr_hbm.at[slab.at[t]], tg.at[b], sg[b])

        def wait_g(t, b):
            pltpu.make_async_copy(
                wr_hbm.at[slab.at[t]], tg.at[b], sg[b]
            ).wait()

        def out_slice(t):
            return out_hbm.at[t, :, pl.ds(128 * wid, 128)]

        def fire_o(t, b):
            pltpu.async_copy(ob.at[b, :, pl.ds(0, 128)], out_slice(t), so[b])

        def wait_o(t, b):
            pltpu.make_async_copy(
                ob.at[b, :, pl.ds(0, 128)], out_slice(t), so[b]
            ).wait()

        def transpose_out(b):
            frows = [_iota16() + 16 * m for m in range(4)]

            @plsc.parallel_loop(0, 128, 1, unroll=8)
            def krow(k):
                kcol = jnp.full((16,), k, jnp.int32)
                for m in range(4):
                    v = tg[b, k, pl.ds(16 * m, 16)]
                    plsc.store_scatter(ob.at[b], [frows[m], kcol], v)

        fire_g(0, 0)
        fire_g(1, 1)
        for b in (0, 1):
            wait_g(b, b)
            transpose_out(b)
            fire_o(b, b)
            fire_g(b + 2, b)

        def body(p, carry):
            for b in (0, 1):
                t = 2 * p + b
                wait_g(t, b)
                wait_o(t - 2, b)
                transpose_out(b)
                fire_o(t, b)
                fire_g(t + 2, b)
            return carry

        lax.fori_loop(1, n_blk // 2 - 1, body, 0)

        for b in (0, 1):
            t = n_blk - 2 + b
            wait_g(t, b)
            wait_o(t - 2, b)
            transpose_out(b)
            fire_o(t, b)
        wait_o(n_blk - 2, 0)
        wait_o(n_blk - 1, 1)

    return k2


def kernel(input_, weight):
    bsz, seq = input_.shape
    wnext = jnp.concatenate([weight[1:], weight[:1]], axis=0)
    wr = jnp.concatenate([weight, wnext], axis=1)
    idx_t = input_.astype(jnp.int32).T
    n_s = idx_t.shape[0]
    out_t = _make_lookup(n_s)(idx_t, wr)
    return out_t.transpose(2, 0, 1)


# speedup vs baseline: 1.0626x; 1.0626x over previous
import functools

import jax
import jax.numpy as jnp
from jax import lax
from jax.experimental import pallas as pl
from jax.experimental.pallas import tpu as pltpu
from jax.experimental.pallas import tpu_sc as plsc

_info = plsc.get_sparse_core_info()
_NC, _NS = _info.num_cores, _info.num_subcores
_NW = _NC * _NS


def _make_gather(B: int, ch: int):
    b_per_w = B // _NW
    n_ch = b_per_w // ch
    assert n_ch % 2 == 0 and n_ch >= 4
    mesh = plsc.VectorSubcoreMesh(core_axis_name="c", subcore_axis_name="s")

    @functools.partial(
        pl.kernel,
        mesh=mesh,
        out_type=jax.ShapeDtypeStruct((B, 128), jnp.float32),
        scratch_types=[
            pltpu.VMEM((n_ch, ch), jnp.int32),
            pltpu.VMEM((2, ch, 128), jnp.float32),
            pltpu.SemaphoreType.DMA,
            pltpu.SemaphoreType.DMA,
            pltpu.SemaphoreType.DMA,
            pltpu.SemaphoreType.DMA,
        ],
        compiler_params=pltpu.CompilerParams(use_tc_tiling_on_sc=False),
    )
    def k(idx_hbm, w_hbm, out_hbm, idx_v, rows_v, sg0, sg1, so0, so1):
        wid = lax.axis_index("s") * _NC + lax.axis_index("c")
        base_w = wid * b_per_w
        sg = (sg0, sg1)
        so = (so0, so1)

        def fire_g(c, b):
            pltpu.async_copy(w_hbm.at[idx_v.at[c]], rows_v.at[b], sg[b])

        def wait_g(c, b):
            pltpu.make_async_copy(
                w_hbm.at[idx_v.at[c]], rows_v.at[b], sg[b]
            ).wait()

        def fire_o(c, b):
            pltpu.async_copy(
                rows_v.at[b], out_hbm.at[pl.ds(base_w + c * ch, ch)], so[b]
            )

        def wait_o(c, b):
            pltpu.make_async_copy(
                rows_v.at[b], out_hbm.at[pl.ds(base_w + c * ch, ch)], so[b]
            ).wait()

        pltpu.sync_copy(idx_hbm.at[wid], idx_v)

        fire_g(0, 0)
        fire_g(1, 1)
        wait_g(0, 0)
        fire_o(0, 0)

        def body(g, carry):
            for (c, b) in ((2 * g + 1, 1), (2 * g + 2, 0)):
                wait_o(c - 1, 1 - b)
                fire_g(c + 1, 1 - b)
                wait_g(c, b)
                fire_o(c, b)
            return carry

        lax.fori_loop(0, (n_ch - 2) // 2, body, 0)

        c = n_ch - 1
        wait_o(c - 1, 0)
        wait_g(c, 1)
        fire_o(c, 1)
        wait_o(c, 1)

    return k


def kernel(input_, weight):
    bsz, seq = input_.shape
    V, D = weight.shape
    wr = weight.reshape(V // 2, 2 * D)
    idx = input_.reshape(-1).astype(jnp.int32)
    B = idx.shape[0]
    ch = 256
    i2 = jnp.right_shift(idx, 1).reshape(_NW, (B // _NW) // ch, ch)
    pairs = _make_gather(B, ch)(i2, wr)
    half = jnp.where(
        (jnp.bitwise_and(idx, 1) == 1)[:, None], pairs[:, D:], pairs[:, :D]
    )
    return half.reshape(bsz, seq, D)


# speedup vs baseline: 1.3784x; 1.2972x over previous
import functools

import jax
import jax.numpy as jnp
from jax import lax
from jax.experimental import pallas as pl
from jax.experimental.pallas import tpu as pltpu
from jax.experimental.pallas import tpu_sc as plsc

_info = plsc.get_sparse_core_info()
_NC, _NS = _info.num_cores, _info.num_subcores
_NW = _NC * _NS


def _make_gather(B: int, V: int, D: int, ch: int):
    b_per_w = B // _NW
    n_ch = b_per_w // ch
    assert n_ch % 2 == 0 and n_ch >= 4
    mesh = plsc.VectorSubcoreMesh(core_axis_name="c", subcore_axis_name="s")

    @functools.partial(
        pl.kernel,
        mesh=mesh,
        out_type=jax.ShapeDtypeStruct((B, D), jnp.float32),
        scratch_types=[
            pltpu.VMEM((n_ch, ch), jnp.int32),
            pltpu.VMEM((2, ch, D), jnp.float32),
            pltpu.SemaphoreType.DMA,
            pltpu.SemaphoreType.DMA,
            pltpu.SemaphoreType.DMA,
            pltpu.SemaphoreType.DMA,
        ],
        compiler_params=pltpu.CompilerParams(use_tc_tiling_on_sc=False),
    )
    def k(idx_hbm, w_hbm, out_hbm, idx_v, rows_v, sg0, sg1, so0, so1):
        wid = lax.axis_index("s") * _NC + lax.axis_index("c")
        base_w = wid * b_per_w
        sg = (sg0, sg1)
        so = (so0, so1)

        def fire_g(c, b):
            pltpu.async_copy(w_hbm.at[idx_v.at[c]], rows_v.at[b], sg[b])

        def wait_g(c, b):
            pltpu.make_async_copy(
                w_hbm.at[idx_v.at[c]], rows_v.at[b], sg[b]
            ).wait()

        def fire_o(c, b):
            pltpu.async_copy(
                rows_v.at[b], out_hbm.at[pl.ds(base_w + c * ch, ch)], so[b]
            )

        def wait_o(c, b):
            pltpu.make_async_copy(
                rows_v.at[b], out_hbm.at[pl.ds(base_w + c * ch, ch)], so[b]
            ).wait()

        pltpu.sync_copy(idx_hbm.at[wid], idx_v)

        fire_g(0, 0)
        fire_g(1, 1)
        wait_g(0, 0)
        fire_o(0, 0)

        def body(g, carry):
            for (c, b) in ((2 * g + 1, 1), (2 * g + 2, 0)):
                wait_o(c - 1, 1 - b)
                fire_g(c + 1, 1 - b)
                wait_g(c, b)
                fire_o(c, b)
            return carry

        lax.fori_loop(0, (n_ch - 2) // 2, body, 0)

        c = n_ch - 1
        wait_o(c - 1, 0)
        wait_g(c, 1)
        fire_o(c, 1)
        wait_o(c, 1)

    return k


def kernel(input_, weight):
    bsz, seq = input_.shape
    V, D = weight.shape
    idx = input_.reshape(-1).astype(jnp.int32)
    B = idx.shape[0]
    ch = 512
    idx3 = idx.reshape(_NW, (B // _NW) // ch, ch)
    out = _make_gather(B, V, D, ch=ch)(idx3, weight)
    return out.reshape(bsz, seq, D)
